# baseline (speedup 1.0000x reference)
import functools

import jax
import jax.numpy as jnp
from jax import lax
from jax.experimental import pallas as pl
from jax.experimental.pallas import tpu as pltpu
from jax.experimental.pallas import tpu_sc as plsc

_N = 1000000
_D = 1000000
_H = 16
_C = 6
_B = 16384

_NC = 2
_NS = 16
_NW = _NC * _NS
_BPW = _B // _NW


def _gather_body(x0_hbm, x1_hbm, u_tab, v_tab, u_out, v_out,
                 idx_u, idx_v, rows_u, rows_v, sem_u, sem_v):
  wid = lax.axis_index("s") * _NC + lax.axis_index("c")
  base = wid * _BPW
  pltpu.sync_copy(x0_hbm.at[pl.ds(base, _BPW)], idx_u)
  pltpu.sync_copy(x1_hbm.at[pl.ds(base, _BPW)], idx_v)
  cu = pltpu.async_copy(u_tab.at[idx_u], rows_u, sem_u)
  cv = pltpu.async_copy(v_tab.at[idx_v], rows_v, sem_v)
  cu.wait()
  cv.wait()
  pltpu.sync_copy(rows_u, u_out.at[pl.ds(base, _BPW)])
  pltpu.sync_copy(rows_v, v_out.at[pl.ds(base, _BPW)])


_sc_gather = pl.kernel(
    _gather_body,
    out_type=(
        jax.ShapeDtypeStruct((_B, _H), jnp.float32),
        jax.ShapeDtypeStruct((_B, _H), jnp.float32),
    ),
    mesh=plsc.VectorSubcoreMesh(core_axis_name="c", subcore_axis_name="s"),
    compiler_params=pltpu.CompilerParams(use_tc_tiling_on_sc=False),
    scratch_types=[
        pltpu.VMEM((_BPW,), jnp.int32),
        pltpu.VMEM((_BPW,), jnp.int32),
        pltpu.VMEM((_BPW, _H), jnp.float32),
        pltpu.VMEM((_BPW, _H), jnp.float32),
        pltpu.SemaphoreType.DMA,
        pltpu.SemaphoreType.DMA,
    ],
)


_MM_BLK = 2048


def _mm_body(u_ref, v_ref, wu_ref, wv_ref, b_ref, o_ref):
  acc = jnp.dot(u_ref[...], wu_ref[...], preferred_element_type=jnp.float32)
  acc += jnp.dot(v_ref[...], wv_ref[...], preferred_element_type=jnp.float32)
  o_ref[...] = acc + b_ref[...]


_tc_matmul = pl.pallas_call(
    _mm_body,
    grid=(_B // _MM_BLK,),
    in_specs=[
        pl.BlockSpec((_MM_BLK, _H), lambda i: (i, 0)),
        pl.BlockSpec((_MM_BLK, _H), lambda i: (i, 0)),
        pl.BlockSpec((_H, _C), lambda i: (0, 0)),
        pl.BlockSpec((_H, _C), lambda i: (0, 0)),
        pl.BlockSpec((1, _C), lambda i: (0, 0)),
    ],
    out_specs=pl.BlockSpec((_MM_BLK, _C), lambda i: (i, 0)),
    out_shape=jax.ShapeDtypeStruct((_B, _C), jnp.float32),
)


@jax.jit
def kernel(X_batch, U, V, W, b):
  x0 = X_batch[:, 0].astype(jnp.int32)
  x1 = X_batch[:, 1].astype(jnp.int32)
  u_rows, v_rows = _sc_gather(x0, x1, U, V)
  wu = W[:_H]
  wv = W[_H:]
  return _tc_matmul(u_rows, v_rows, wu, wv, b.reshape(1, _C))
